# baseline (speedup 1.0000x reference)
import functools

import numpy as np
import jax
import jax.numpy as jnp
from jax import lax
from jax.experimental import pallas as pl
from jax.experimental.pallas import tpu as pltpu
from jax.experimental.pallas import tpu_sc as plsc

RADIAL_FEATURES = 32
CHANNELS = 128
CUTOFF = 5.0
N_NODES = 10000
N_EDGES = 160000
MAX_Z = 119

_NC, _NS, _L = 2, 16, 16
_NW = _NC * _NS
_E_PER = 5008
_E_PAD = _E_PER * _NW


def _sc_gather(src_hbm, dst_hbm, rflat_hbm, z_hbm,
               dx_hbm, dy_hbm, dz_hbm, zs_hbm, zd_hbm,
               src_v, dst_v, r_v, z_v, dxv, dyv, dzv, zsv, zdv):
    wid = lax.axis_index("s") * _NC + lax.axis_index("c")
    base = wid * _E_PER
    pltpu.sync_copy(src_hbm.at[pl.ds(base, _E_PER)], src_v)
    pltpu.sync_copy(dst_hbm.at[pl.ds(base, _E_PER)], dst_v)
    pltpu.sync_copy(rflat_hbm, r_v)
    pltpu.sync_copy(z_hbm, z_v)

    def body(i, carry):
        off = i * _L
        s = src_v[pl.ds(off, _L)]
        d = dst_v[pl.ds(off, _L)]
        s3 = s * 3
        d3 = d * 3
        xs = plsc.load_gather(r_v, [s3])
        ys = plsc.load_gather(r_v, [s3 + 1])
        zs = plsc.load_gather(r_v, [s3 + 2])
        xd = plsc.load_gather(r_v, [d3])
        yd = plsc.load_gather(r_v, [d3 + 1])
        zd = plsc.load_gather(r_v, [d3 + 2])
        zn_s = plsc.load_gather(z_v, [s]).astype(jnp.float32)
        zn_d = plsc.load_gather(z_v, [d]).astype(jnp.float32)
        dxv[pl.ds(off, _L)] = xd - xs
        dyv[pl.ds(off, _L)] = yd - ys
        dzv[pl.ds(off, _L)] = zd - zs
        zsv[pl.ds(off, _L)] = zn_s
        zdv[pl.ds(off, _L)] = zn_d
        return carry

    lax.fori_loop(0, _E_PER // _L, body, 0)
    sl = pl.ds(base, _E_PER)
    pltpu.sync_copy(dxv, dx_hbm.at[sl])
    pltpu.sync_copy(dyv, dy_hbm.at[sl])
    pltpu.sync_copy(dzv, dz_hbm.at[sl])
    pltpu.sync_copy(zsv, zs_hbm.at[sl])
    pltpu.sync_copy(zdv, zd_hbm.at[sl])


def _sc_gather_call(src, dst, rflat, z32):
    col = jax.ShapeDtypeStruct((_E_PAD,), jnp.float32)
    wrapped = functools.partial(
        pl.kernel,
        mesh=plsc.VectorSubcoreMesh(core_axis_name="c", subcore_axis_name="s"),
        out_type=(col, col, col, col, col),
        scratch_types=[
            pltpu.VMEM((_E_PER,), jnp.int32),
            pltpu.VMEM((_E_PER,), jnp.int32),
            pltpu.VMEM((N_NODES * 3,), jnp.float32),
            pltpu.VMEM((N_NODES,), jnp.int32),
            pltpu.VMEM((_E_PER,), jnp.float32),
            pltpu.VMEM((_E_PER,), jnp.float32),
            pltpu.VMEM((_E_PER,), jnp.float32),
            pltpu.VMEM((_E_PER,), jnp.float32),
            pltpu.VMEM((_E_PER,), jnp.float32),
        ],
        compiler_params=pltpu.CompilerParams(needs_layout_passes=False),
    )(_sc_gather)
    return wrapped(src, dst, rflat, z32)


_BETA = float(((2.0 / RADIAL_FEATURES) * (1.0 - np.exp(-CUTOFF))) ** -2)

_MU = np.linspace(np.exp(-CUTOFF), 1.0, RADIAL_FEATURES).astype(np.float32)

_ENV_C = (1.0, -0.24999994, 0.020833245, -0.00069439015,
          1.2384941e-05, -1.3539515e-07, 8.622546e-10)

_B = 1600


def _tc_body(dx_ref, dy_ref, dz_ref, zs_ref, zd_ref,
             zemb_ref, wz_ref, wr_ref, br_ref, mu_ref,
             oi_ref, oa_ref, os_ref):
    dx = dx_ref[...]
    dy = dy_ref[...]
    dz = dz_ref[...]
    r2 = dx * dx + dy * dy + dz * dz
    r = jnp.sqrt(r2)
    inv = lax.rsqrt(r2)
    x = dx * inv
    y = dy * inv
    z = dz * inv

    er = jnp.exp(-r)
    diff = er - mu_ref[...]
    feats = jnp.exp(-_BETA * diff * diff)
    hr = jnp.dot(feats, wr_ref[...], preferred_element_type=jnp.float32)
    hr = hr + br_ref[...]
    v = r2 * jnp.float32((np.pi / CUTOFF) ** 2)
    p = jnp.float32(_ENV_C[-1])
    for _c in _ENV_C[-2::-1]:
        p = p * v + jnp.float32(_c)
    env = jnp.where(r2 < CUTOFF * CUTOFF, p, 0.0)

    zs = zs_ref[...].astype(jnp.int32)
    zd = zd_ref[...].astype(jnp.int32)
    ii = lax.broadcasted_iota(jnp.int32, (_B, CHANNELS), 1)
    oh_s = (zs == ii).astype(jnp.float32)
    oh_d = (zd == ii).astype(jnp.float32)
    zemb = zemb_ref[...]
    e_s = jnp.dot(oh_s, zemb, preferred_element_type=jnp.float32)
    e_d = jnp.dot(oh_d, zemb, preferred_element_type=jnp.float32)
    wz = wz_ref[...]
    hz = (jnp.dot(e_s, wz[0:CHANNELS, :], preferred_element_type=jnp.float32)
          + jnp.dot(e_d, wz[CHANNELS:, :], preferred_element_type=jnp.float32))

    shape = (_B, CHANNELS)
    envb = jnp.broadcast_to(env, shape)
    xb = jnp.broadcast_to(x, shape)
    yb = jnp.broadcast_to(y, shape)
    zb = jnp.broadcast_to(z, shape)
    t3 = (x * x + y * y + z * z) / 3.0
    t3b = jnp.broadcast_to(t3, shape)

    ehz = envb * hz
    c_i = hr[:, 0:CHANNELS] * ehz
    c_a = hr[:, CHANNELS:2 * CHANNELS] * ehz
    c_s = hr[:, 2 * CHANNELS:] * ehz

    zero = jnp.zeros(shape, jnp.float32)

    oi_ref[0] = c_i
    oi_ref[1] = zero
    oi_ref[2] = zero
    oi_ref[3] = zero
    oi_ref[4] = c_i
    oi_ref[5] = zero
    oi_ref[6] = zero
    oi_ref[7] = zero
    oi_ref[8] = c_i

    pax = xb * c_a
    pay = yb * c_a
    paz = zb * c_a
    oa_ref[0] = zero
    oa_ref[1] = -paz
    oa_ref[2] = pay
    oa_ref[3] = paz
    oa_ref[4] = zero
    oa_ref[5] = -pax
    oa_ref[6] = -pay
    oa_ref[7] = pax
    oa_ref[8] = zero

    tx = xb * c_s
    ty = yb * c_s
    tz = zb * c_s
    t3s = t3b * c_s
    pxy = xb * ty
    pxz = xb * tz
    pyz = yb * tz
    os_ref[0] = xb * tx - t3s
    os_ref[1] = pxy
    os_ref[2] = pxz
    os_ref[3] = pxy
    os_ref[4] = yb * ty - t3s
    os_ref[5] = pyz
    os_ref[6] = pxz
    os_ref[7] = pyz
    os_ref[8] = zb * tz - t3s


def _tc_call(cols, zemb, wz, wr, br, mu):
    n_blocks = N_EDGES // _B
    full = lambda i: (0, 0)
    colspec = pl.BlockSpec((_B, 1), lambda i: (i, 0))
    out_shape = jax.ShapeDtypeStruct((9, N_EDGES, CHANNELS), jnp.float32)
    oplane = pl.BlockSpec((9, _B, CHANNELS), lambda i: (0, i, 0))
    return pl.pallas_call(
        _tc_body,
        grid=(n_blocks,),
        in_specs=[
            colspec, colspec, colspec, colspec, colspec,
            pl.BlockSpec((CHANNELS, CHANNELS), full),
            pl.BlockSpec((2 * CHANNELS, CHANNELS), full),
            pl.BlockSpec((RADIAL_FEATURES, 3 * CHANNELS), full),
            pl.BlockSpec((1, 3 * CHANNELS), full),
            pl.BlockSpec((1, RADIAL_FEATURES), full),
        ],
        out_specs=[oplane, oplane, oplane],
        out_shape=[out_shape, out_shape, out_shape],
        compiler_params=pltpu.CompilerParams(
            dimension_semantics=("arbitrary",),
            vmem_limit_bytes=100 * 1024 * 1024),
    )(*cols, zemb, wz, wr, br, mu)


def _planes_to_out(p, E):
    return jnp.transpose(p.reshape(3, 3, E, CHANNELS), (2, 3, 0, 1))


def kernel(Z, edge_index, R, z_embed, W_z, W_r, b_r):
    E = edge_index.shape[1]
    pad = _E_PAD - E
    src = jnp.concatenate([edge_index[0].astype(jnp.int32),
                           jnp.zeros((pad,), jnp.int32)])
    dst = jnp.concatenate([edge_index[1].astype(jnp.int32),
                           jnp.zeros((pad,), jnp.int32)])
    rflat = R.astype(jnp.float32).reshape(-1)
    z32 = Z.astype(jnp.int32)

    cols = [c.reshape(_E_PAD, 1) for c in _sc_gather_call(src, dst, rflat, z32)]

    zemb = jnp.zeros((CHANNELS, CHANNELS), jnp.float32).at[:MAX_Z, :].set(z_embed)
    br2 = b_r.reshape(1, 3 * CHANNELS)
    mu2 = jnp.asarray(_MU).reshape(1, RADIAL_FEATURES)
    oi, oa, os_ = _tc_call(cols, zemb, W_z, W_r, br2, mu2)
    return (_planes_to_out(oi, E), _planes_to_out(oa, E),
            _planes_to_out(os_, E))


# speedup vs baseline: 1.4900x; 1.4900x over previous
import functools

import numpy as np
import jax
import jax.numpy as jnp
from jax import lax
from jax.experimental import pallas as pl
from jax.experimental.pallas import tpu as pltpu
from jax.experimental.pallas import tpu_sc as plsc

RADIAL_FEATURES = 32
CHANNELS = 128
CUTOFF = 5.0
N_NODES = 10000
N_EDGES = 160000
MAX_Z = 119

_NC, _NS, _L = 2, 16, 16
_NW = _NC * _NS
_E_PER = 5008
_E_PAD = _E_PER * _NW


def _sc_gather(src_hbm, dst_hbm, rflat_hbm, z_hbm, out_hbm,
               src_v, dst_v, r_v, z_v, out_v):
    wid = lax.axis_index("s") * _NC + lax.axis_index("c")
    base = wid * _E_PER
    pltpu.sync_copy(src_hbm.at[pl.ds(base, _E_PER)], src_v)
    pltpu.sync_copy(dst_hbm.at[pl.ds(base, _E_PER)], dst_v)
    pltpu.sync_copy(rflat_hbm, r_v)
    pltpu.sync_copy(z_hbm, z_v)

    lane = lax.iota(jnp.int32, _L)

    def body(i, carry):
        off = i * _L
        s = src_v[pl.ds(off, _L)]
        d = dst_v[pl.ds(off, _L)]
        s3 = s * 3
        d3 = d * 3
        xs = plsc.load_gather(r_v, [s3])
        ys = plsc.load_gather(r_v, [s3 + 1])
        zs = plsc.load_gather(r_v, [s3 + 2])
        xd = plsc.load_gather(r_v, [d3])
        yd = plsc.load_gather(r_v, [d3 + 1])
        zd = plsc.load_gather(r_v, [d3 + 2])
        zn_s = plsc.load_gather(z_v, [s]).astype(jnp.float32)
        zn_d = plsc.load_gather(z_v, [d]).astype(jnp.float32)
        o = (off + lane) * 8
        plsc.store_scatter(out_v, [o], xd - xs)
        plsc.store_scatter(out_v, [o + 1], yd - ys)
        plsc.store_scatter(out_v, [o + 2], zd - zs)
        plsc.store_scatter(out_v, [o + 4], zn_s)
        plsc.store_scatter(out_v, [o + 5], zn_d)
        return carry

    lax.fori_loop(0, _E_PER // _L, body, 0)
    pltpu.sync_copy(out_v, out_hbm.at[pl.ds(base * 8, _E_PER * 8)])


def _sc_gather_call(src, dst, rflat, z32):
    wrapped = functools.partial(
        pl.kernel,
        mesh=plsc.VectorSubcoreMesh(core_axis_name="c", subcore_axis_name="s"),
        out_type=jax.ShapeDtypeStruct((_E_PAD * 8,), jnp.float32),
        scratch_types=[
            pltpu.VMEM((_E_PER,), jnp.int32),
            pltpu.VMEM((_E_PER,), jnp.int32),
            pltpu.VMEM((N_NODES * 3,), jnp.float32),
            pltpu.VMEM((N_NODES,), jnp.int32),
            pltpu.VMEM((_E_PER * 8,), jnp.float32),
        ],
        compiler_params=pltpu.CompilerParams(needs_layout_passes=False),
    )(_sc_gather)
    return wrapped(src, dst, rflat, z32)


_BETA = float(((2.0 / RADIAL_FEATURES) * (1.0 - np.exp(-CUTOFF))) ** -2)

_MU = np.linspace(np.exp(-CUTOFF), 1.0, RADIAL_FEATURES).astype(np.float32)

_ENV_C = (1.0, -0.24999994, 0.020833245, -0.00069439015,
          1.2384941e-05, -1.3539515e-07, 8.622546e-10)

_B = 1600


def _tc_body(rv_ref, zemb_ref, wz_ref, wr_ref, br_ref, mu_ref,
             oi_ref, oa_ref, os_ref):
    rv = rv_ref[...]
    dx = rv[:, 0:1]
    dy = rv[:, 1:2]
    dz = rv[:, 2:3]
    r2 = dx * dx + dy * dy + dz * dz
    r = jnp.sqrt(r2)
    inv = lax.rsqrt(r2)
    x = dx * inv
    y = dy * inv
    z = dz * inv

    er = jnp.exp(-r)
    diff = er - mu_ref[...]
    feats = jnp.exp(-_BETA * diff * diff)
    hr = jnp.dot(feats, wr_ref[...], preferred_element_type=jnp.float32)
    hr = hr + br_ref[...]
    v = r2 * jnp.float32((np.pi / CUTOFF) ** 2)
    p = jnp.float32(_ENV_C[-1])
    for _c in _ENV_C[-2::-1]:
        p = p * v + jnp.float32(_c)
    env = jnp.where(r2 < CUTOFF * CUTOFF, p, 0.0)

    zs = rv[:, 4:5].astype(jnp.int32)
    zd = rv[:, 5:6].astype(jnp.int32)
    ii = lax.broadcasted_iota(jnp.int32, (_B, CHANNELS), 1)
    oh_s = (zs == ii).astype(jnp.float32)
    oh_d = (zd == ii).astype(jnp.float32)
    zemb = zemb_ref[...]
    e_s = jnp.dot(oh_s, zemb, preferred_element_type=jnp.float32)
    e_d = jnp.dot(oh_d, zemb, preferred_element_type=jnp.float32)
    wz = wz_ref[...]
    hz = (jnp.dot(e_s, wz[0:CHANNELS, :], preferred_element_type=jnp.float32)
          + jnp.dot(e_d, wz[CHANNELS:, :], preferred_element_type=jnp.float32))

    shape = (_B, CHANNELS)
    envb = jnp.broadcast_to(env, shape)
    xb = jnp.broadcast_to(x, shape)
    yb = jnp.broadcast_to(y, shape)
    zb = jnp.broadcast_to(z, shape)
    t3 = (x * x + y * y + z * z) / 3.0
    t3b = jnp.broadcast_to(t3, shape)

    ehz = envb * hz
    c_i = hr[:, 0:CHANNELS] * ehz
    c_a = hr[:, CHANNELS:2 * CHANNELS] * ehz
    c_s = hr[:, 2 * CHANNELS:] * ehz

    zero = jnp.zeros(shape, jnp.float32)

    oi_ref[0] = c_i
    oi_ref[1] = zero
    oi_ref[2] = zero
    oi_ref[3] = zero
    oi_ref[4] = c_i
    oi_ref[5] = zero
    oi_ref[6] = zero
    oi_ref[7] = zero
    oi_ref[8] = c_i

    pax = xb * c_a
    pay = yb * c_a
    paz = zb * c_a
    oa_ref[0] = zero
    oa_ref[1] = -paz
    oa_ref[2] = pay
    oa_ref[3] = paz
    oa_ref[4] = zero
    oa_ref[5] = -pax
    oa_ref[6] = -pay
    oa_ref[7] = pax
    oa_ref[8] = zero

    tx = xb * c_s
    ty = yb * c_s
    tz = zb * c_s
    t3s = t3b * c_s
    pxy = xb * ty
    pxz = xb * tz
    pyz = yb * tz
    os_ref[0] = xb * tx - t3s
    os_ref[1] = pxy
    os_ref[2] = pxz
    os_ref[3] = pxy
    os_ref[4] = yb * ty - t3s
    os_ref[5] = pyz
    os_ref[6] = pxz
    os_ref[7] = pyz
    os_ref[8] = zb * tz - t3s


def _tc_call(rv8, zemb, wz, wr, br, mu):
    n_blocks = N_EDGES // _B
    full = lambda i: (0, 0)
    out_shape = jax.ShapeDtypeStruct((9, N_EDGES, CHANNELS), jnp.float32)
    oplane = pl.BlockSpec((9, _B, CHANNELS), lambda i: (0, i, 0))
    return pl.pallas_call(
        _tc_body,
        grid=(n_blocks,),
        in_specs=[
            pl.BlockSpec((_B, 8), lambda i: (i, 0)),
            pl.BlockSpec((CHANNELS, CHANNELS), full),
            pl.BlockSpec((2 * CHANNELS, CHANNELS), full),
            pl.BlockSpec((RADIAL_FEATURES, 3 * CHANNELS), full),
            pl.BlockSpec((1, 3 * CHANNELS), full),
            pl.BlockSpec((1, RADIAL_FEATURES), full),
        ],
        out_specs=[oplane, oplane, oplane],
        out_shape=[out_shape, out_shape, out_shape],
        compiler_params=pltpu.CompilerParams(
            dimension_semantics=("arbitrary",),
            vmem_limit_bytes=100 * 1024 * 1024),
    )(rv8, zemb, wz, wr, br, mu)


def _planes_to_out(p, E):
    return jnp.transpose(p.reshape(3, 3, E, CHANNELS), (2, 3, 0, 1))


def kernel(Z, edge_index, R, z_embed, W_z, W_r, b_r):
    E = edge_index.shape[1]
    pad = _E_PAD - E
    src = jnp.concatenate([edge_index[0].astype(jnp.int32),
                           jnp.zeros((pad,), jnp.int32)])
    dst = jnp.concatenate([edge_index[1].astype(jnp.int32),
                           jnp.zeros((pad,), jnp.int32)])
    rflat = R.astype(jnp.float32).reshape(-1)
    z32 = Z.astype(jnp.int32)

    rv8 = _sc_gather_call(src, dst, rflat, z32).reshape(_E_PAD, 8)

    zemb = jnp.zeros((CHANNELS, CHANNELS), jnp.float32).at[:MAX_Z, :].set(z_embed)
    br2 = b_r.reshape(1, 3 * CHANNELS)
    mu2 = jnp.asarray(_MU).reshape(1, RADIAL_FEATURES)
    oi, oa, os_ = _tc_call(rv8, zemb, W_z, W_r, br2, mu2)
    return (_planes_to_out(oi, E), _planes_to_out(oa, E),
            _planes_to_out(os_, E))
